# Initial kernel scaffold; baseline (speedup 1.0000x reference)
import jax
import jax.numpy as jnp
from jax.experimental import pallas as pl


def kernel(x, edge_index, W1, b1, W2, b2):
    raise NotImplementedError("write your pallas kernel here")


# speedup vs baseline: 10.9435x; 10.9435x over previous
import functools

import jax
import jax.numpy as jnp
from jax import lax
from jax.experimental import pallas as pl
from jax.experimental.pallas import tpu as pltpu
from jax.experimental.pallas import tpu_sc as plsc

_N = 10000
_E = 160000
_DIN = 256
_DHID = 512
_NCLS = 40
_NPAD = 128

_NCORES = 2
_NSUB = 16
_LANES = 128

_A1_CHUNKS = 79
_A1_PER_SUB = _A1_CHUNKS * _LANES

_A2_CHUNKS = 40
_A2_PER_SUB = _A2_CHUNKS * _LANES

_NROW = 10112
_TRASH = 10000
_STRIPE = _NROW // _NSUB

_GRID = 5
_ROWBLK = _N // _GRID

_MESH = plsc.VectorSubcoreMesh(core_axis_name="core", subcore_axis_name="subcore")


def _sc_hist(dst_blk, ones16, z16):
    @functools.partial(
        pl.kernel,
        out_type=jax.ShapeDtypeStruct((_NCORES, _NROW, 128), jnp.float32),
        mesh=_MESH,
        scratch_types=[
            pltpu.VMEM((_A2_CHUNKS, _LANES), jnp.int32),
            pltpu.VMEM((_LANES, 128), jnp.float32),
            pltpu.VMEM_SHARED((_NROW, 128), jnp.float32),
        ],
    )
    def k(dst_hbm, ones_hbm, z_hbm, out_hbm, dst_v, ones_v, hist_sh):
        c = lax.axis_index("core")
        s = lax.axis_index("subcore")
        w = c * _NSUB + s
        pltpu.sync_copy(z_hbm.at[pl.ds(s * _STRIPE, _STRIPE)],
                        hist_sh.at[pl.ds(s * _STRIPE, _STRIPE)])
        pltpu.sync_copy(dst_hbm.at[w], dst_v)
        pltpu.sync_copy(ones_hbm, ones_v)
        plsc.subcore_barrier()

        @pl.loop(0, _A2_CHUNKS)
        def _(j):
            pltpu.sync_copy(ones_v, hist_sh.at[dst_v.at[j]], add=True)

        plsc.subcore_barrier()
        pltpu.sync_copy(hist_sh.at[pl.ds(s * _STRIPE, _STRIPE)],
                        out_hbm.at[c, pl.ds(s * _STRIPE, _STRIPE)])

    return k(dst_blk, ones16, z16)


def _sc_agg1(x2, idx_blk, dst_blk, z128):
    @functools.partial(
        pl.kernel,
        out_type=jax.ShapeDtypeStruct((_NCORES, _NROW, 128), jnp.float32),
        mesh=_MESH,
        scratch_types=[
            pltpu.VMEM((_A1_CHUNKS, _LANES), jnp.int32),
            pltpu.VMEM((_A1_CHUNKS, _LANES), jnp.int32),
            pltpu.VMEM((_LANES, 128), jnp.float32),
            pltpu.VMEM_SHARED((_NROW, 128), jnp.float32),
        ],
    )
    def k(x2_hbm, idx_hbm, dst_hbm, z_hbm, out_hbm, idx_v, dst_v, rows_v, acc_sh):
        c = lax.axis_index("core")
        s = lax.axis_index("subcore")
        w = c * _NSUB + s
        pltpu.sync_copy(z_hbm.at[pl.ds(s * _STRIPE, _STRIPE)],
                        acc_sh.at[pl.ds(s * _STRIPE, _STRIPE)])
        pltpu.sync_copy(idx_hbm.at[w], idx_v)
        pltpu.sync_copy(dst_hbm.at[s], dst_v)
        plsc.subcore_barrier()

        @pl.loop(0, _A1_CHUNKS)
        def _(j):
            pltpu.sync_copy(x2_hbm.at[idx_v.at[j]], rows_v)
            pltpu.sync_copy(rows_v, acc_sh.at[dst_v.at[j]], add=True)

        plsc.subcore_barrier()
        pltpu.sync_copy(acc_sh.at[pl.ds(s * _STRIPE, _STRIPE)],
                        out_hbm.at[c, pl.ds(s * _STRIPE, _STRIPE)])

    return k(x2, idx_blk, dst_blk, z128)


def _sc_agg2(ps, src_blk, dst_blk, z64):
    @functools.partial(
        pl.kernel,
        out_type=jax.ShapeDtypeStruct((_NCORES, _NROW, _NPAD), jnp.float32),
        mesh=_MESH,
        scratch_types=[
            pltpu.VMEM((_A2_CHUNKS, _LANES), jnp.int32),
            pltpu.VMEM((_A2_CHUNKS, _LANES), jnp.int32),
            pltpu.VMEM((_LANES, _NPAD), jnp.float32),
            pltpu.VMEM_SHARED((_NROW, _NPAD), jnp.float32),
        ],
    )
    def k(ps_hbm, idx_hbm, dst_hbm, z_hbm, out_hbm, idx_v, dst_v, rows_v, acc_sh):
        c = lax.axis_index("core")
        s = lax.axis_index("subcore")
        w = c * _NSUB + s
        pltpu.sync_copy(z_hbm.at[pl.ds(s * _STRIPE, _STRIPE)],
                        acc_sh.at[pl.ds(s * _STRIPE, _STRIPE)])
        pltpu.sync_copy(idx_hbm.at[w], idx_v)
        pltpu.sync_copy(dst_hbm.at[w], dst_v)
        plsc.subcore_barrier()

        @pl.loop(0, _A2_CHUNKS)
        def _(j):
            pltpu.sync_copy(ps_hbm.at[idx_v.at[j]], rows_v)
            pltpu.sync_copy(rows_v, acc_sh.at[dst_v.at[j]], add=True)

        plsc.subcore_barrier()
        pltpu.sync_copy(acc_sh.at[pl.ds(s * _STRIPE, _STRIPE)],
                        out_hbm.at[c, pl.ds(s * _STRIPE, _STRIPE)])

    return k(ps, src_blk, dst_blk, z64)


def _dinv_of(hp_ref):
    deg = hp_ref[0, :, 0:1] + hp_ref[1, :, 0:1] + 1.0
    return lax.rsqrt(deg)


def _tc_scale(x, histp):
    def body(hp_ref, x_ref, o_ref):
        o_ref[...] = x_ref[...] * _dinv_of(hp_ref)

    return pl.pallas_call(
        body,
        grid=(_GRID,),
        in_specs=[
            pl.BlockSpec((_NCORES, _ROWBLK, 128), lambda i: (0, i, 0)),
            pl.BlockSpec((_ROWBLK, _DIN), lambda i: (i, 0)),
        ],
        out_specs=pl.BlockSpec((_ROWBLK, _DIN), lambda i: (i, 0)),
        out_shape=jax.ShapeDtypeStruct((_N, _DIN), jnp.float32),
    )(histp, x)


def _tc_mlp(xs, agg1, histp, W1, b1, W2p):
    def body(hp_ref, xs_ref, a_ref, w1_ref, b1_ref, w2_ref, o_ref):
        dinv = _dinv_of(hp_ref)
        aggc = jnp.concatenate([a_ref[0], a_ref[1]], axis=1)
        y = (xs_ref[...] + aggc) * dinv
        h = jnp.dot(y, w1_ref[...], preferred_element_type=jnp.float32)
        h = jnp.maximum(h + b1_ref[...], 0.0)
        p = jnp.dot(h, w2_ref[...], preferred_element_type=jnp.float32)
        o_ref[...] = p * dinv

    return pl.pallas_call(
        body,
        grid=(_GRID,),
        in_specs=[
            pl.BlockSpec((_NCORES, _ROWBLK, 128), lambda i: (0, i, 0)),
            pl.BlockSpec((_ROWBLK, _DIN), lambda i: (i, 0)),
            pl.BlockSpec((_NCORES, _ROWBLK, 128), lambda i: (0, i, 0)),
            pl.BlockSpec((_DIN, _DHID), lambda i: (0, 0)),
            pl.BlockSpec((1, _DHID), lambda i: (0, 0)),
            pl.BlockSpec((_DHID, _NPAD), lambda i: (0, 0)),
        ],
        out_specs=pl.BlockSpec((_ROWBLK, _NPAD), lambda i: (i, 0)),
        out_shape=jax.ShapeDtypeStruct((_N, _NPAD), jnp.float32),
    )(histp, xs, agg1, W1, b1, W2p)


def _tc_final(ps, agg2, histp, b2p):
    def body(hp_ref, ps_ref, a_ref, b2_ref, o_ref):
        dinv = _dinv_of(hp_ref)
        z = (ps_ref[...] + a_ref[0] + a_ref[1]) * dinv + b2_ref[...]
        col = lax.broadcasted_iota(jnp.int32, (_ROWBLK, _NPAD), 1)
        mask = col < _NCLS
        zm = jnp.where(mask, z, -jnp.inf)
        m = jnp.max(zm, axis=1, keepdims=True)
        e = jnp.where(mask, jnp.exp(z - m), 0.0)
        lse = jnp.log(jnp.sum(e, axis=1, keepdims=True))
        o_ref[...] = z - m - lse

    return pl.pallas_call(
        body,
        grid=(_GRID,),
        in_specs=[
            pl.BlockSpec((_NCORES, _ROWBLK, 128), lambda i: (0, i, 0)),
            pl.BlockSpec((_ROWBLK, _NPAD), lambda i: (i, 0)),
            pl.BlockSpec((_NCORES, _ROWBLK, _NPAD), lambda i: (0, i, 0)),
            pl.BlockSpec((1, _NPAD), lambda i: (0, 0)),
        ],
        out_specs=pl.BlockSpec((_ROWBLK, _NPAD), lambda i: (i, 0)),
        out_shape=jax.ShapeDtypeStruct((_N, _NPAD), jnp.float32),
    )(histp, ps, agg2, b2p)


def kernel(x, edge_index, W1, b1, W2, b2):
    src = edge_index[0].astype(jnp.int32)
    dst = edge_index[1].astype(jnp.int32)

    e1 = _E // _NSUB
    s1 = jnp.pad(src.reshape(_NSUB, e1), ((0, 0), (0, _A1_PER_SUB - e1)))
    d1 = jnp.pad(dst.reshape(_NSUB, e1), ((0, 0), (0, _A1_PER_SUB - e1)),
                 constant_values=_TRASH)
    idx1 = jnp.concatenate([2 * s1, 2 * s1 + 1], axis=0)
    idx1 = idx1.reshape(_NCORES * _NSUB, _A1_CHUNKS, _LANES)
    dst1 = d1.reshape(_NSUB, _A1_CHUNKS, _LANES)

    e2 = _E // (_NCORES * _NSUB)
    s2 = jnp.pad(src.reshape(_NCORES * _NSUB, e2), ((0, 0), (0, _A2_PER_SUB - e2)))
    d2 = jnp.pad(dst.reshape(_NCORES * _NSUB, e2), ((0, 0), (0, _A2_PER_SUB - e2)),
                 constant_values=_TRASH)
    src2 = s2.reshape(_NCORES * _NSUB, _A2_CHUNKS, _LANES)
    dst2 = d2.reshape(_NCORES * _NSUB, _A2_CHUNKS, _LANES)

    ones128 = jnp.ones((_LANES, 128), jnp.float32)
    z128 = jnp.zeros((_NROW, 128), jnp.float32)
    W2p = jnp.pad(W2, ((0, 0), (0, _NPAD - _NCLS)))
    b2p = jnp.pad(b2, (0, _NPAD - _NCLS)).reshape(1, _NPAD)

    histp = _sc_hist(dst2, ones128, z128)[:, :_N]
    xs = _tc_scale(x, histp)
    agg1 = _sc_agg1(xs.reshape(2 * _N, 128), idx1, dst1, z128)[:, :_N]
    ps = _tc_mlp(xs, agg1, histp, W1, b1.reshape(1, _DHID), W2p)
    agg2 = _sc_agg2(ps, src2, dst2, z128)[:, :_N]
    zfull = _tc_final(ps, agg2, histp, b2p)
    return zfull[:, :_NCLS]
